# baseline (speedup 1.0000x reference)
import functools

import jax
import jax.numpy as jnp
from jax import lax
from jax.experimental import pallas as pl
from jax.experimental.pallas import tpu as pltpu
from jax.experimental.pallas import tpu_sc as plsc

_L = 16
_NC = 2
_B = 16
_S = 2048
_D = 1024


def _sc_body(seq_hbm, mask_hbm, out_hbm, mask_v, row_v, sem1, sem2):
    wid = lax.axis_index("s")

    @pl.when(wid < _B)
    def _():
        b = wid
        _H = _S // 2
        cp1 = pltpu.make_async_copy(
            mask_hbm.at[b, pl.ds(0, _H)], mask_v.at[pl.ds(0, _H)], sem1)
        cp2 = pltpu.make_async_copy(
            mask_hbm.at[b, pl.ds(_H, _H)], mask_v.at[pl.ds(_H, _H)], sem2)
        cp1.start()
        cp2.start()

        def _sum_step(i, acc):
            return acc + mask_v[pl.ds(i * _L, _L)]

        cp1.wait()
        acc = lax.fori_loop(0, _H // _L, _sum_step,
                            jnp.zeros((_L,), jnp.int32), unroll=8)
        cp2.wait()
        acc = lax.fori_loop(_H // _L, _S // _L, _sum_step, acc, unroll=8)
        total = acc[0]
        for lane in range(1, _L):
            total = total + acc[lane]
        idx = jnp.maximum(total - 1, 0)

        mv = mask_v[pl.ds(idx, _L)][0]

        @pl.when(mv != 0)
        def _copy_row():
            pltpu.sync_copy(seq_hbm.at[b, idx], out_hbm.at[b])

        @pl.when(mv == 0)
        def _zero_row():
            z = jnp.zeros((_L,), jnp.float32)
            for i in range(_D // _L):
                row_v[pl.ds(i * _L, _L)] = z
            pltpu.sync_copy(row_v, out_hbm.at[b])


@jax.jit
def kernel(sequence, mask):
    mesh = plsc.VectorSubcoreMesh(core_axis_name="c", subcore_axis_name="s",
                                  num_cores=1)
    fn = pl.kernel(
        _sc_body,
        mesh=mesh,
        out_type=jax.ShapeDtypeStruct((_B, _D), jnp.float32),
        scratch_types=[
            pltpu.VMEM((_S + _L,), jnp.int32),
            pltpu.VMEM((_D,), jnp.float32),
            pltpu.SemaphoreType.DMA,
            pltpu.SemaphoreType.DMA,
        ],
    )
    return fn(sequence, mask)


# speedup vs baseline: 1.0636x; 1.0636x over previous
import functools

import jax
import jax.numpy as jnp
from jax import lax
from jax.experimental import pallas as pl
from jax.experimental.pallas import tpu as pltpu
from jax.experimental.pallas import tpu_sc as plsc

_L = 16
_NC = 2
_B = 16
_S = 2048
_D = 1024


def _sc_body(seq_hbm, mask_hbm, out_hbm, mask_v, row_v, sem1, sem2):
    wid = lax.axis_index("s")

    @pl.when(wid < _B)
    def _():
        b = wid
        _H = _S // 2
        cp1 = pltpu.make_async_copy(
            mask_hbm.at[b, pl.ds(0, _H)], mask_v.at[pl.ds(0, _H)], sem1)
        cp2 = pltpu.make_async_copy(
            mask_hbm.at[b, pl.ds(_H, _H)], mask_v.at[pl.ds(_H, _H)], sem2)
        cp1.start()
        cp2.start()

        def _sum_step(i, acc):
            return acc + mask_v[pl.ds(i * _L, _L)]

        cp1.wait()
        acc = lax.fori_loop(0, _H // _L, _sum_step,
                            jnp.zeros((_L,), jnp.int32), unroll=8)
        cp2.wait()
        acc = lax.fori_loop(_H // _L, _S // _L, _sum_step, acc, unroll=8)
        total = acc[0]
        for lane in range(1, _L):
            total = total + acc[lane]
        idx = jnp.maximum(total - 1, 0)

        mv = mask_v[pl.ds(idx, _L)][0]

        @pl.when(mv != 0)
        def _copy_row():
            pltpu.sync_copy(seq_hbm.at[b, idx], row_v)

        @pl.when(mv == 0)
        def _zero_row():
            z = jnp.zeros((_L,), jnp.float32)
            for i in range(_D // _L):
                row_v[pl.ds(i * _L, _L)] = z

        pltpu.sync_copy(row_v, out_hbm.at[b])


@jax.jit
def kernel(sequence, mask):
    mesh = plsc.VectorSubcoreMesh(core_axis_name="c", subcore_axis_name="s",
                                  num_cores=1)
    fn = pl.kernel(
        _sc_body,
        mesh=mesh,
        out_type=jax.ShapeDtypeStruct((_B, _D), jnp.float32),
        scratch_types=[
            pltpu.VMEM((_S + _L,), jnp.int32),
            pltpu.VMEM((_D,), jnp.float32),
            pltpu.SemaphoreType.DMA,
            pltpu.SemaphoreType.DMA,
        ],
    )
    return fn(sequence, mask)


# speedup vs baseline: 1.0641x; 1.0004x over previous
import functools

import jax
import jax.numpy as jnp
from jax import lax
from jax.experimental import pallas as pl
from jax.experimental.pallas import tpu as pltpu
from jax.experimental.pallas import tpu_sc as plsc

_L = 16
_NC = 2
_B = 16
_S = 2048
_D = 1024


def _sc_body(seq_hbm, mask_hbm, out_hbm, mask_v, row_v):
    b = lax.axis_index("s")
    pltpu.sync_copy(mask_hbm.at[b], mask_v.at[pl.ds(0, _S)])

    accs = [jnp.zeros((_L,), jnp.int32) for _ in range(4)]
    for i in range(0, _S // _L, 4):
        for j in range(4):
            accs[j] = accs[j] + mask_v[pl.ds((i + j) * _L, _L)]
    acc = (accs[0] + accs[1]) + (accs[2] + accs[3])
    total = acc[0]
    for lane in range(1, _L):
        total = total + acc[lane]
    idx = jnp.maximum(total - 1, 0)

    mv = mask_v[pl.ds(idx, _L)][0]

    @pl.when(mv != 0)
    def _copy_row():
        pltpu.sync_copy(seq_hbm.at[b, idx], row_v)

    @pl.when(mv == 0)
    def _zero_row():
        z = jnp.zeros((_L,), jnp.float32)
        for i in range(_D // _L):
            row_v[pl.ds(i * _L, _L)] = z

    pltpu.sync_copy(row_v, out_hbm.at[b])


@jax.jit
def kernel(sequence, mask):
    mesh = plsc.VectorSubcoreMesh(core_axis_name="c", subcore_axis_name="s",
                                  num_cores=1)
    fn = pl.kernel(
        _sc_body,
        mesh=mesh,
        out_type=jax.ShapeDtypeStruct((_B, _D), jnp.float32),
        scratch_types=[
            pltpu.VMEM((_S + _L,), jnp.int32),
            pltpu.VMEM((_D,), jnp.float32),
        ],
    )
    return fn(sequence, mask)


# speedup vs baseline: 1.0756x; 1.0109x over previous
import functools

import jax
import jax.numpy as jnp
from jax import lax
from jax.experimental import pallas as pl
from jax.experimental.pallas import tpu as pltpu
from jax.experimental.pallas import tpu_sc as plsc

_L = 16
_NC = 2
_B = 16
_S = 2048
_D = 1024


def _sc_body(seq_hbm, mask_hbm, out_hbm, mask_v, row_v):
    b = lax.axis_index("s")
    pltpu.sync_copy(mask_hbm.at[b], mask_v.at[pl.ds(0, _S)])

    def _sum_step(i, acc):
        return acc + mask_v[pl.ds(i * _L, _L)]

    acc = lax.fori_loop(0, _S // _L, _sum_step,
                        jnp.zeros((_L,), jnp.int32), unroll=8)
    total = acc[0]
    for lane in range(1, _L):
        total = total + acc[lane]
    idx = jnp.maximum(total - 1, 0)

    mv = mask_v[pl.ds(idx, _L)][0]

    @pl.when(mv != 0)
    def _copy_row():
        pltpu.sync_copy(seq_hbm.at[b, idx], row_v)

    @pl.when(mv == 0)
    def _zero_row():
        z = jnp.zeros((_L,), jnp.float32)

        def _z_step(i, c):
            row_v[pl.ds(i * _L, _L)] = z
            return c

        lax.fori_loop(0, _D // _L, _z_step, 0, unroll=4)

    pltpu.sync_copy(row_v, out_hbm.at[b])


@jax.jit
def kernel(sequence, mask):
    mesh = plsc.VectorSubcoreMesh(core_axis_name="c", subcore_axis_name="s",
                                  num_cores=1)
    fn = pl.kernel(
        _sc_body,
        mesh=mesh,
        out_type=jax.ShapeDtypeStruct((_B, _D), jnp.float32),
        scratch_types=[
            pltpu.VMEM((_S + _L,), jnp.int32),
            pltpu.VMEM((_D,), jnp.float32),
        ],
    )
    return fn(sequence, mask)
